# Initial kernel scaffold; baseline (speedup 1.0000x reference)
import jax
import jax.numpy as jnp
from jax.experimental import pallas as pl


def kernel(x, w_dw1, w_pw1, scale1, bias1, w_dw2, w_pw2, scale2, bias2, w_res):
    raise NotImplementedError("write your pallas kernel here")


# speedup vs baseline: 2.1649x; 2.1649x over previous
import functools

import jax
import jax.numpy as jnp
from jax.experimental import pallas as pl
from jax.experimental.pallas import tpu as pltpu

_E = 2


def _down3d_block_kernel(x_ref, wdw1_ref, s1_ref, b1_ref, m1_ref,
                         wdw2_ref, s2_ref, b2_ref, m2_ref, mres_ref,
                         o_ref, xp_ref, y1_ref, *, NB, D, H, W, Cin, Cout):
    e = _E
    Wc = W * Cin
    bf16 = jnp.bfloat16

    xp_ref[...] = jnp.zeros_like(xp_ref)
    xp_ref[:, e:e + D, e:e + H, pl.ds(e * Cin, Wc)] = x_ref[...]

    def dw27(src_ref, w_ref):
        acc = None
        for kd in range(3):
            for kh in range(3):
                for kw in range(3):
                    tap = src_ref[:, kd * e:kd * e + D, kh * e:kh * e + H,
                                  pl.ds(kw * e * Cin, Wc)]
                    t = tap * w_ref[kd, kh * 3 + kw]
                    acc = t if acc is None else acc + t
        return acc

    a1 = dw27(xp_ref, wdw1_ref)
    z1 = jnp.clip(a1 * s1_ref[0] + b1_ref[0], 0.0, 6.0)
    y1 = jnp.dot(z1.reshape(NB * D * H, Wc).astype(bf16), m1_ref[...],
                 preferred_element_type=jnp.float32)

    y1_ref[...] = jnp.zeros_like(y1_ref)
    y1_ref[:, e:e + D, e:e + H, pl.ds(e * Cin, Wc)] = y1.reshape(NB, D, H, Wc)

    a2 = dw27(y1_ref, wdw2_ref)
    z2 = jnp.clip(a2 * s2_ref[0] + b2_ref[0], 0.0, 6.0)
    r2 = jnp.dot(z2.reshape(NB * D * H, Wc).astype(bf16), m2_ref[...],
                 preferred_element_type=jnp.float32)

    xi = x_ref[...].reshape(NB * D * H, Wc).astype(bf16)
    rr = jnp.dot(xi, mres_ref[...], preferred_element_type=jnp.float32)

    o_ref[...] = (r2 + rr).reshape(NB, D, H, W * Cout)


def _down3d(xt, wdw1, s1, b1, m1, wdw2, s2, b2, m2, mres,
            *, NB, D, H, W, Cin, Cout):
    N = xt.shape[0]
    e = _E
    Wc, Wco = W * Cin, W * Cout
    Dh, Hh, Whc = D + 2 * e, H + 2 * e, (W + 2 * e) * Cin

    kern = functools.partial(_down3d_block_kernel, NB=NB, D=D, H=H, W=W,
                             Cin=Cin, Cout=Cout)
    zero2 = lambda i: (0, 0)
    zero3 = lambda i: (0, 0, 0)

    return pl.pallas_call(
        kern,
        out_shape=jax.ShapeDtypeStruct((N, D, H, Wco), jnp.float32),
        grid=(N // NB,),
        in_specs=[
            pl.BlockSpec((NB, D, H, Wc), lambda i: (i, 0, 0, 0)),
            pl.BlockSpec((3, 9, Wc), zero3),
            pl.BlockSpec((1, Wc), zero2),
            pl.BlockSpec((1, Wc), zero2),
            pl.BlockSpec((Wc, Wc), zero2),
            pl.BlockSpec((3, 9, Wc), zero3),
            pl.BlockSpec((1, Wc), zero2),
            pl.BlockSpec((1, Wc), zero2),
            pl.BlockSpec((Wc, Wco), zero2),
            pl.BlockSpec((Wc, Wco), zero2),
        ],
        out_specs=pl.BlockSpec((NB, D, H, Wco), lambda i: (i, 0, 0, 0)),
        scratch_shapes=[
            pltpu.VMEM((NB, Dh, Hh, Whc), jnp.float32),
            pltpu.VMEM((NB, Dh, Hh, Whc), jnp.float32),
        ],
        compiler_params=pltpu.CompilerParams(
            dimension_semantics=("parallel",),
            vmem_limit_bytes=48 << 20),
    )(xt, wdw1, s1, b1, m1, wdw2, s2, b2, m2, mres)


def kernel(x, w_dw1, w_pw1, scale1, bias1, w_dw2, w_pw2, scale2, bias2,
           w_res):
    N, Cin, D, H, W = x.shape
    Cout = w_res.shape[1]
    f32, bf16 = jnp.float32, jnp.bfloat16
    NB = 2

    xt = jnp.transpose(x.astype(f32), (0, 2, 3, 4, 1)).reshape(N, D, H, W * Cin)

    wdw1 = jnp.tile(w_dw1.astype(f32), (1, W)).reshape(3, 9, W * Cin)
    wdw2 = jnp.tile(w_dw2.astype(f32), (1, W)).reshape(3, 9, W * Cin)
    s1 = jnp.tile(scale1.reshape(1, Cin).astype(f32), (1, W))
    b1 = jnp.tile(bias1.reshape(1, Cin).astype(f32), (1, W))
    s2 = jnp.tile(scale2.reshape(1, Cin).astype(f32), (1, W))
    b2 = jnp.tile(bias2.reshape(1, Cin).astype(f32), (1, W))

    eye = jnp.eye(W, dtype=f32)
    m1 = jnp.kron(eye, w_pw1.astype(f32)).astype(bf16)
    m2 = jnp.kron(eye, w_pw2.astype(f32)).astype(bf16)
    mres = jnp.kron(eye, w_res.astype(f32)).astype(bf16)

    out = _down3d(xt, wdw1, s1, b1, m1, wdw2, s2, b2, m2, mres,
                  NB=NB, D=D, H=H, W=W, Cin=Cin, Cout=Cout)
    out = out.reshape(N, D, H, W, Cout)
    return jnp.transpose(out, (0, 4, 1, 2, 3))
